# baseline (speedup 1.0000x reference)
import jax
import jax.numpy as jnp
from jax.experimental import pallas as pl

_N_NODES = 100000
_NUM_GRAPHS = 512
_RRELU_SLOPE = (1.0 / 8.0 + 1.0 / 3.0) / 2.0


def _rrelu(x):
    return jnp.where(x >= 0, x, x * _RRELU_SLOPE)


def _mlp_body(g_ref, w1_ref, b1_ref, w2_ref, b2_ref, w3_ref, b3_ref, out_ref):
    g = g_ref[...]
    g = _rrelu(g + jnp.dot(g, w1_ref[...], preferred_element_type=jnp.float32) + b1_ref[...])
    g = _rrelu(g + jnp.dot(g, w2_ref[...], preferred_element_type=jnp.float32) + b2_ref[...])
    g = _rrelu(jnp.dot(g, w3_ref[...], preferred_element_type=jnp.float32) + b3_ref[...])
    out_ref[...] = g


def _mlp(g, lin1_W, lin1_b, lin2_W, lin2_b, lin3_W, lin3_b):
    return pl.pallas_call(
        _mlp_body,
        out_shape=jax.ShapeDtypeStruct((_NUM_GRAPHS, 1), jnp.float32),
    )(g, lin1_W, lin1_b.reshape(1, -1), lin2_W, lin2_b.reshape(1, -1),
      lin3_W, lin3_b.reshape(1, -1))


def _gcn_conv(x, src, dst, W, b, edge_weight=None):
    n = x.shape[0]
    if edge_weight is None:
        edge_weight = jnp.ones((src.shape[0],), dtype=x.dtype)
    deg = jax.ops.segment_sum(edge_weight, dst, num_segments=n)
    dinv = jnp.where(deg > 0, jax.lax.rsqrt(jnp.where(deg > 0, deg, 1.0)), 0.0)
    norm = dinv[src] * edge_weight * dinv[dst]
    xw = x @ W
    msg = norm[:, None] * jnp.take(xw, src, axis=0)
    out = jax.ops.segment_sum(msg, dst, num_segments=n)
    return out + b


def kernel(x, edge_index, batch, edge_weights, W1, b1, W2, b2,
           lin1_W, lin1_b, lin2_W, lin2_b, lin3_W, lin3_b):
    src = edge_index[0]
    dst = edge_index[1]
    h = _gcn_conv(x, src, dst, W1, b1, edge_weight=edge_weights)
    agg = jax.ops.segment_max(jnp.take(h, src, axis=0), dst, num_segments=h.shape[0])
    h = jnp.maximum(h, agg)
    res = h
    h = _gcn_conv(h, src, dst, W2, b2, edge_weight=None)
    h = jax.nn.relu(res + h)
    g = jax.ops.segment_max(h, batch, num_segments=_NUM_GRAPHS)
    return _mlp(g, lin1_W, lin1_b, lin2_W, lin2_b, lin3_W, lin3_b)


# speedup vs baseline: 9.9591x; 9.9591x over previous
import functools

import jax
import jax.numpy as jnp
from jax import lax
from jax.experimental import pallas as pl
from jax.experimental.pallas import tpu as pltpu
from jax.experimental.pallas import tpu_sc as plsc

_N = 100000
_E = 3200000
_NG = 512
_NW = 32
_BW = 3200
_NPAD = _NW * _BW
_NLOC = _BW + 16
_EC = _E // _NW
_EPAD = _E + _NW * 128
_RRELU_SLOPE = (1.0 / 8.0 + 1.0 / 3.0) / 2.0
_CNT_ADJ = 1
_INCREMENT_A = False

_MESH = plsc.VectorSubcoreMesh(core_axis_name="c", subcore_axis_name="s")


def _wid():
    return lax.axis_index("s") * 2 + lax.axis_index("c")


def _bucket_of(dv):
    return ((dv >> 7) * 2622) >> 16


_Z16 = lambda: jnp.zeros((16,), jnp.int32)


def _p1_hist(dst):
    @functools.partial(
        pl.kernel,
        out_type=jax.ShapeDtypeStruct((_NW, 32), jnp.int32),
        mesh=_MESH,
        compiler_params=pltpu.CompilerParams(needs_layout_passes=False, use_tc_tiling_on_sc=False),
        scratch_types=[
            pltpu.VMEM((2000,), jnp.int32),
            pltpu.VMEM((32,), jnp.int32),
        ],
    )
    def k(dst_h, hist_h, dbuf, histv):
        t = _wid()
        histv[pl.ds(0, 16)] = _Z16()
        histv[pl.ds(16, 16)] = _Z16()

        def chunk(c, _):
            pltpu.sync_copy(dst_h.at[pl.ds(t * _EC + c * 2000, 2000)], dbuf)

            def grp(g, _):
                bv = _bucket_of(dbuf[pl.ds(g * 16, 16)])
                cnt, lastm = plsc.scan_count(bv)
                hv = plsc.load_gather(histv, [bv])
                plsc.store_scatter(histv, [bv], hv + cnt + (1 - _CNT_ADJ),
                                   mask=lastm)
                return _
            lax.fori_loop(0, 125, grp, None)
            return _
        lax.fori_loop(0, _EC // 2000, chunk, None)
        pltpu.sync_copy(histv, hist_h.at[t])

    return k(dst)


def _p2_partition(src, dst, w, hist):
    @functools.partial(
        pl.kernel,
        out_type=(
            jax.ShapeDtypeStruct((_EPAD,), jnp.int32),
            jax.ShapeDtypeStruct((_EPAD,), jnp.int32),
            jax.ShapeDtypeStruct((_EPAD,), jnp.float32),
            jax.ShapeDtypeStruct((_NW, 16), jnp.int32),
        ),
        mesh=_MESH,
        compiler_params=pltpu.CompilerParams(needs_layout_passes=False, use_tc_tiling_on_sc=False),
        scratch_types=[
            pltpu.VMEM((_NW, 32), jnp.int32),
            pltpu.VMEM((80,), jnp.int32),
            pltpu.VMEM((80,), jnp.int32),
            pltpu.VMEM((80,), jnp.float32),
            pltpu.VMEM((80,), jnp.int32),
            pltpu.VMEM((32,), jnp.int32),
            pltpu.VMEM((16,), jnp.int32),
            pltpu.SemaphoreType.DMA,
        ],
    )
    def k(src_h, dst_h, w_h, hist_h, srcP, dstP, wP, meta_h,
          hbuf, sbuf, dbuf, wbuf, posb, curv_ref, mbuf, sem):
        t = _wid()
        pltpu.sync_copy(hist_h, hbuf)
        z = _Z16()

        def csum(tt, carry):
            lo, hi = carry
            return (lo + hbuf[tt, pl.ds(0, 16)], hi + hbuf[tt, pl.ds(16, 16)])
        cs_lo, cs_hi = lax.fori_loop(0, 32, csum, (z, z))

        pad_lo = ((cs_lo + 127) >> 7) << 7
        pad_hi = ((cs_hi + 127) >> 7) << 7
        inc_lo = plsc.cumsum(pad_lo)
        inc_hi = plsc.cumsum(pad_hi)
        off_lo = inc_lo - pad_lo
        off_hi = inc_hi - pad_hi + inc_lo[15]

        def psum(tt, carry):
            lo, hi = carry
            take = tt < t
            lo = lo + jnp.where(take, hbuf[tt, pl.ds(0, 16)], z)
            hi = hi + jnp.where(take, hbuf[tt, pl.ds(16, 16)], z)
            return (lo, hi)
        p_lo, p_hi = lax.fori_loop(0, 32, psum, (z, z))

        curv_ref[pl.ds(0, 16)] = off_lo + p_lo
        curv_ref[pl.ds(16, 16)] = off_hi + p_hi

        i16 = lax.iota(jnp.int32, 16)
        offt = (jnp.sum(jnp.where(i16 == t, off_lo, z))
                + jnp.sum(jnp.where(i16 == t - 16, off_hi, z)))
        cntt = (jnp.sum(jnp.where(i16 == t, cs_lo, z))
                + jnp.sum(jnp.where(i16 == t - 16, cs_hi, z)))
        mbuf[...] = jnp.where(i16 == 0, offt, z) + jnp.where(i16 == 1, cntt, z)
        pltpu.sync_copy(mbuf, meta_h.at[t])

        def chunk(c, _):
            s0 = t * _EC + c * 80
            pltpu.sync_copy(src_h.at[pl.ds(s0, 80)], sbuf)
            pltpu.sync_copy(dst_h.at[pl.ds(s0, 80)], dbuf)
            pltpu.sync_copy(w_h.at[pl.ds(s0, 80)], wbuf)

            for g in range(5):
                bv = _bucket_of(dbuf[pl.ds(g * 16, 16)])
                cnt, lastm = plsc.scan_count(bv)
                curv = plsc.load_gather(curv_ref, [bv])
                posv = curv + cnt - _CNT_ADJ
                plsc.store_scatter(curv_ref, [bv], posv + 1, mask=lastm)
                posb[pl.ds(g * 16, 16)] = jnp.clip(posv, 0, _EPAD - 1)

            c1 = pltpu.async_copy(sbuf, srcP.at[posb], sem)
            c2 = pltpu.async_copy(dbuf, dstP.at[posb], sem)
            c3 = pltpu.async_copy(wbuf, wP.at[posb], sem)
            c1.wait()
            c2.wait()
            c3.wait()
            return _
        lax.fori_loop(0, _EC // 80, chunk, None)

    return k(src, dst, w, hist)


def _p25_deg(dstP, wP, meta):
    @functools.partial(
        pl.kernel,
        out_type=(
            jax.ShapeDtypeStruct((_NPAD, 16), jnp.float32),
            jax.ShapeDtypeStruct((_NPAD, 16), jnp.float32),
        ),
        mesh=_MESH,
        compiler_params=pltpu.CompilerParams(needs_layout_passes=False, use_tc_tiling_on_sc=False),
        scratch_types=[
            pltpu.VMEM((16,), jnp.int32),
            pltpu.VMEM((128,), jnp.int32),
            pltpu.VMEM((128,), jnp.float32),
            pltpu.VMEM((_NLOC, 16), jnp.float32),
            pltpu.VMEM((_NLOC, 16), jnp.float32),
        ],
    )
    def k(dstP, wP, meta_h, deg1_h, deg2_h, mbuf, dbuf, wbuf, d1, d2):
        t = _wid()
        base = t * _BW
        pltpu.sync_copy(meta_h.at[t], mbuf)
        mv = mbuf[...]
        off = pl.multiple_of(mv[0], 128)
        cnt = mv[1]

        zf = jnp.zeros((16,), jnp.float32)
        onef = jnp.ones((16,), jnp.float32)

        def zrow(i, _):
            d1[i] = zf
            d2[i] = zf
            return _
        lax.fori_loop(0, _NLOC, zrow, None)

        i16 = lax.iota(jnp.int32, 16)
        nch = (cnt + 127) >> 7

        def chunk(c, _):
            s0 = off + c * 128
            pltpu.sync_copy(dstP.at[pl.ds(s0, 128)], dbuf)
            pltpu.sync_copy(wP.at[pl.ds(s0, 128)], wbuf)

            def grp(g, _):
                m = (c * 128 + g * 16 + i16) < cnt
                dl = jnp.clip(dbuf[pl.ds(g * 16, 16)] - base, 0, _BW)
                dl = jnp.where(m, dl, _BW)
                wv = jnp.where(m, wbuf[pl.ds(g * 16, 16)], 0.0)
                for i in range(16):
                    d = dl[i]
                    plsc.addupdate(d1.at[d], jnp.full((16,), 1.0) * wv[i])
                    plsc.addupdate(d2.at[d], onef)
                return _
            lax.fori_loop(0, 8, grp, None)
            return _
        lax.fori_loop(0, nch, chunk, None)

        pltpu.sync_copy(d1.at[pl.ds(0, _BW)], deg1_h.at[pl.ds(base, _BW)])
        pltpu.sync_copy(d2.at[pl.ds(0, _BW)], deg2_h.at[pl.ds(base, _BW)])

    return k(dstP, wP, meta)


def _p3_conv1(srcP, dstP, wP, meta, xsc, dinv1, b1):
    @functools.partial(
        pl.kernel,
        out_type=jax.ShapeDtypeStruct((_NPAD, 16), jnp.float32),
        mesh=_MESH,
        compiler_params=pltpu.CompilerParams(needs_layout_passes=False, use_tc_tiling_on_sc=False),
        scratch_types=[
            pltpu.VMEM((16,), jnp.int32),
            pltpu.VMEM((128,), jnp.int32),
            pltpu.VMEM((128,), jnp.int32),
            pltpu.VMEM((128,), jnp.float32),
            pltpu.VMEM((128,), jnp.int32),
            pltpu.VMEM((128, 16), jnp.float32),
            pltpu.VMEM((_NLOC, 16), jnp.float32),
            pltpu.VMEM((_BW,), jnp.float32),
            pltpu.VMEM((16,), jnp.float32),
            pltpu.SemaphoreType.DMA,
        ],
    )
    def k(srcP, dstP, wP, meta_h, xsc_h, dinv1_h, b1_h, h1_h,
          mbuf, sbuf, dbuf, wbuf, ibuf, rows, acc, dloc, b1v, sem):
        t = _wid()
        base = t * _BW
        pltpu.sync_copy(meta_h.at[t], mbuf)
        mv = mbuf[...]
        off = pl.multiple_of(mv[0], 128)
        cnt = mv[1]
        pltpu.sync_copy(dinv1_h.at[pl.ds(base, _BW)], dloc)
        pltpu.sync_copy(b1_h, b1v)

        zf = jnp.zeros((16,), jnp.float32)

        def zrow(i, _):
            acc[i] = zf
            return _
        lax.fori_loop(0, _NLOC, zrow, None)

        i16 = lax.iota(jnp.int32, 16)
        nch = (cnt + 127) >> 7

        def chunk(c, _):
            s0 = off + c * 128
            pltpu.sync_copy(srcP.at[pl.ds(s0, 128)], sbuf)
            pltpu.sync_copy(dstP.at[pl.ds(s0, 128)], dbuf)
            pltpu.sync_copy(wP.at[pl.ds(s0, 128)], wbuf)

            def gmask(g, _):
                m = (c * 128 + g * 16 + i16) < cnt
                ibuf[pl.ds(g * 16, 16)] = jnp.clip(
                    jnp.where(m, sbuf[pl.ds(g * 16, 16)], 0), 0, _N - 1)
                return _
            lax.fori_loop(0, 8, gmask, None)
            pltpu.async_copy(xsc_h.at[ibuf], rows, sem).wait()

            def grp(g, _):
                m = (c * 128 + g * 16 + i16) < cnt
                dl = jnp.where(m, jnp.clip(dbuf[pl.ds(g * 16, 16)] - base, 0, _BW), _BW)
                wv = jnp.where(m, wbuf[pl.ds(g * 16, 16)], 0.0)
                for i in range(16):
                    plsc.addupdate(acc.at[dl[i]], rows[g * 16 + i] * wv[i])
                return _
            lax.fori_loop(0, 8, grp, None)
            return _
        lax.fori_loop(0, nch, chunk, None)

        b1vv = b1v[...]

        def nrow(j, _):
            dv = dloc[pl.ds(j * 16, 16)]
            for i in range(16):
                r = j * 16 + i
                acc[r] = acc[r] * dv[i] + b1vv
            return _
        lax.fori_loop(0, _BW // 16, nrow, None)
        pltpu.sync_copy(acc.at[pl.ds(0, _BW)], h1_h.at[pl.ds(base, _BW)])

    return k(srcP, dstP, wP, meta, xsc, dinv1, b1)


def _p4_maxpool(srcP, dstP, meta, h1):
    @functools.partial(
        pl.kernel,
        out_type=jax.ShapeDtypeStruct((_NPAD, 16), jnp.float32),
        mesh=_MESH,
        compiler_params=pltpu.CompilerParams(needs_layout_passes=False, use_tc_tiling_on_sc=False),
        scratch_types=[
            pltpu.VMEM((16,), jnp.int32),
            pltpu.VMEM((128,), jnp.int32),
            pltpu.VMEM((128,), jnp.int32),
            pltpu.VMEM((128,), jnp.int32),
            pltpu.VMEM((128, 16), jnp.float32),
            pltpu.VMEM((_NLOC, 16), jnp.float32),
            pltpu.VMEM((_BW, 16), jnp.float32),
            pltpu.SemaphoreType.DMA,
        ],
    )
    def k(srcP, dstP, meta_h, h1_h, hmax_h,
          mbuf, sbuf, dbuf, ibuf, rows, acc, hown, sem):
        t = _wid()
        base = t * _BW
        pltpu.sync_copy(meta_h.at[t], mbuf)
        mv = mbuf[...]
        off = pl.multiple_of(mv[0], 128)
        cnt = mv[1]
        pltpu.sync_copy(h1_h.at[pl.ds(base, _BW)], hown)

        ninf = jnp.full((16,), -jnp.inf, jnp.float32)

        def irow(i, _):
            acc[i] = ninf
            return _
        lax.fori_loop(0, _NLOC, irow, None)

        i16 = lax.iota(jnp.int32, 16)
        nch = (cnt + 127) >> 7

        def chunk(c, _):
            s0 = off + c * 128
            pltpu.sync_copy(srcP.at[pl.ds(s0, 128)], sbuf)
            pltpu.sync_copy(dstP.at[pl.ds(s0, 128)], dbuf)

            def gmask(g, _):
                m = (c * 128 + g * 16 + i16) < cnt
                ibuf[pl.ds(g * 16, 16)] = jnp.clip(
                    jnp.where(m, sbuf[pl.ds(g * 16, 16)], 0), 0, _N - 1)
                return _
            lax.fori_loop(0, 8, gmask, None)
            pltpu.async_copy(h1_h.at[ibuf], rows, sem).wait()

            def grp(g, _):
                m = (c * 128 + g * 16 + i16) < cnt
                dl = jnp.where(m, jnp.clip(dbuf[pl.ds(g * 16, 16)] - base, 0, _BW), _BW)
                for i in range(16):
                    d = dl[i]
                    acc[d] = jnp.maximum(acc[d], rows[g * 16 + i])
                return _
            lax.fori_loop(0, 8, grp, None)
            return _
        lax.fori_loop(0, nch, chunk, None)

        def nrow(r, _):
            acc[r] = jnp.maximum(acc[r], hown[r])
            return _
        lax.fori_loop(0, _BW, nrow, None)
        pltpu.sync_copy(acc.at[pl.ds(0, _BW)], hmax_h.at[pl.ds(base, _BW)])

    return k(srcP, dstP, meta, h1)


def _p5_conv2(srcP, dstP, meta, hsc, hmax, dinv2, b2, batch_pad):
    @functools.partial(
        pl.kernel,
        out_type=jax.ShapeDtypeStruct((_NW, _NG, 16), jnp.float32),
        mesh=_MESH,
        compiler_params=pltpu.CompilerParams(needs_layout_passes=False, use_tc_tiling_on_sc=False),
        scratch_types=[
            pltpu.VMEM((16,), jnp.int32),
            pltpu.VMEM((128,), jnp.int32),
            pltpu.VMEM((128,), jnp.int32),
            pltpu.VMEM((128,), jnp.int32),
            pltpu.VMEM((128, 16), jnp.float32),
            pltpu.VMEM((_NLOC, 16), jnp.float32),
            pltpu.VMEM((_BW, 16), jnp.float32),
            pltpu.VMEM((_BW,), jnp.float32),
            pltpu.VMEM((_BW,), jnp.int32),
            pltpu.VMEM((_NG + 16, 16), jnp.float32),
            pltpu.VMEM((16,), jnp.float32),
            pltpu.SemaphoreType.DMA,
        ],
    )
    def k(srcP, dstP, meta_h, hsc_h, hmax_h, dinv2_h, b2_h, batch_h, gpart_h,
          mbuf, sbuf, dbuf, ibuf, rows, acc, hown, d2loc, bloc, gacc, b2v, sem):
        t = _wid()
        base = t * _BW
        pltpu.sync_copy(meta_h.at[t], mbuf)
        mv = mbuf[...]
        off = pl.multiple_of(mv[0], 128)
        cnt = mv[1]
        pltpu.sync_copy(hmax_h.at[pl.ds(base, _BW)], hown)
        pltpu.sync_copy(dinv2_h.at[pl.ds(base, _BW)], d2loc)
        pltpu.sync_copy(batch_h.at[pl.ds(base, _BW)], bloc)
        pltpu.sync_copy(b2_h, b2v)

        zf = jnp.zeros((16,), jnp.float32)
        ninf = jnp.full((16,), -jnp.inf, jnp.float32)

        def zrow(i, _):
            acc[i] = zf
            return _
        lax.fori_loop(0, _NLOC, zrow, None)

        def irow(i, _):
            gacc[i] = ninf
            return _
        lax.fori_loop(0, _NG + 16, irow, None)

        i16 = lax.iota(jnp.int32, 16)
        nch = (cnt + 127) >> 7

        def chunk(c, _):
            s0 = off + c * 128
            pltpu.sync_copy(srcP.at[pl.ds(s0, 128)], sbuf)
            pltpu.sync_copy(dstP.at[pl.ds(s0, 128)], dbuf)

            def gmask(g, _):
                m = (c * 128 + g * 16 + i16) < cnt
                ibuf[pl.ds(g * 16, 16)] = jnp.clip(
                    jnp.where(m, sbuf[pl.ds(g * 16, 16)], 0), 0, _N - 1)
                return _
            lax.fori_loop(0, 8, gmask, None)
            pltpu.async_copy(hsc_h.at[ibuf], rows, sem).wait()

            def grp(g, _):
                m = (c * 128 + g * 16 + i16) < cnt
                dl = jnp.where(m, jnp.clip(dbuf[pl.ds(g * 16, 16)] - base, 0, _BW), _BW)
                for i in range(16):
                    plsc.addupdate(acc.at[dl[i]], rows[g * 16 + i])
                return _
            lax.fori_loop(0, 8, grp, None)
            return _
        lax.fori_loop(0, nch, chunk, None)

        b2vv = b2v[...]
        nreal = jnp.minimum(_BW, _N - base)

        def nrow(j, _):
            dv = d2loc[pl.ds(j * 16, 16)]
            bv = bloc[pl.ds(j * 16, 16)]
            for i in range(16):
                r = j * 16 + i
                h2 = jnp.maximum(hown[r] + acc[r] * dv[i] + b2vv, 0.0)
                gi = jnp.where(r < nreal, jnp.clip(bv[i], 0, _NG - 1), _NG)
                gacc[gi] = jnp.maximum(gacc[gi], h2)
            return _
        lax.fori_loop(0, _BW // 16, nrow, None)
        pltpu.sync_copy(gacc.at[pl.ds(0, _NG)], gpart_h.at[t])

    return k(srcP, dstP, meta, hsc, hmax, dinv2, b2, batch_pad)


def _dinv(d):
    return jnp.where(d > 0, lax.rsqrt(jnp.where(d > 0, d, 1.0)), 0.0)


def _k1_body(x_ref, d1_ref, d2_ref, w1_ref, xsc_ref, di1_ref, di2_ref):
    di1 = _dinv(d1_ref[:, 0:1])
    di2 = _dinv(d2_ref[:, 0:1])
    xw = jnp.dot(x_ref[...], w1_ref[...], preferred_element_type=jnp.float32)
    xsc_ref[...] = di1 * xw
    di1_ref[...] = di1
    di2_ref[...] = di2


def _k1_scale(xpad, deg1, deg2, W1p):
    blk = 2048
    grid = _NPAD // blk
    return pl.pallas_call(
        _k1_body,
        grid=(grid,),
        in_specs=[
            pl.BlockSpec((blk, 8), lambda i: (i, 0)),
            pl.BlockSpec((blk, 16), lambda i: (i, 0)),
            pl.BlockSpec((blk, 16), lambda i: (i, 0)),
            pl.BlockSpec((8, 16), lambda i: (0, 0)),
        ],
        out_specs=[
            pl.BlockSpec((blk, 16), lambda i: (i, 0)),
            pl.BlockSpec((blk, 1), lambda i: (i, 0)),
            pl.BlockSpec((blk, 1), lambda i: (i, 0)),
        ],
        out_shape=[
            jax.ShapeDtypeStruct((_NPAD, 16), jnp.float32),
            jax.ShapeDtypeStruct((_NPAD, 1), jnp.float32),
            jax.ShapeDtypeStruct((_NPAD, 1), jnp.float32),
        ],
    )(xpad, deg1, deg2, W1p)


def _k2_body(h_ref, di2_ref, w2_ref, hsc_ref):
    hw = jnp.dot(h_ref[...], w2_ref[...], preferred_element_type=jnp.float32)
    hsc_ref[...] = di2_ref[...] * hw


def _k2_scale(hmax, dinv2c, W2):
    blk = 2048
    grid = _NPAD // blk
    return pl.pallas_call(
        _k2_body,
        grid=(grid,),
        in_specs=[
            pl.BlockSpec((blk, 16), lambda i: (i, 0)),
            pl.BlockSpec((blk, 1), lambda i: (i, 0)),
            pl.BlockSpec((16, 16), lambda i: (0, 0)),
        ],
        out_specs=pl.BlockSpec((blk, 16), lambda i: (i, 0)),
        out_shape=jax.ShapeDtypeStruct((_NPAD, 16), jnp.float32),
    )(hmax, dinv2c, W2)


def _rrelu(x):
    return jnp.where(x >= 0, x, x * _RRELU_SLOPE)


def _k3_body(gp_ref, w1_ref, b1_ref, w2_ref, b2_ref, w3_ref, b3_ref, out_ref):
    g = jnp.max(gp_ref[...], axis=0)
    g = _rrelu(g + jnp.dot(g, w1_ref[...], preferred_element_type=jnp.float32) + b1_ref[...])
    g = _rrelu(g + jnp.dot(g, w2_ref[...], preferred_element_type=jnp.float32) + b2_ref[...])
    g = _rrelu(jnp.dot(g, w3_ref[...], preferred_element_type=jnp.float32) + b3_ref[...])
    out_ref[...] = g


def _k3_mlp(gpart, lin1_W, lin1_b, lin2_W, lin2_b, lin3_W, lin3_b):
    return pl.pallas_call(
        _k3_body,
        out_shape=jax.ShapeDtypeStruct((_NG, 1), jnp.float32),
    )(gpart, lin1_W, lin1_b.reshape(1, -1), lin2_W, lin2_b.reshape(1, -1),
      lin3_W, lin3_b.reshape(1, -1))


def kernel(x, edge_index, batch, edge_weights, W1, b1, W2, b2,
           lin1_W, lin1_b, lin2_W, lin2_b, lin3_W, lin3_b):
    src = edge_index[0]
    dst = edge_index[1]

    hist = _p1_hist(dst)
    srcP, dstP, wP, meta = _p2_partition(src, dst, edge_weights, hist)
    deg1p, deg2p = _p25_deg(dstP, wP, meta)

    xpad = jnp.zeros((_NPAD, 8), jnp.float32).at[:_N, :7].set(x)
    W1p = jnp.zeros((8, 16), jnp.float32).at[:7].set(W1)
    xsc, dinv1c, dinv2c = _k1_scale(xpad, deg1p, deg2p, W1p)
    dinv1 = dinv1c.reshape(_NPAD)
    dinv2 = dinv2c.reshape(_NPAD)

    if _INCREMENT_A:
        d1v = dinv1[:_N]
        d2v = dinv2[:_N]
        xw = x @ W1
        norm1 = d1v[src] * edge_weights * d1v[dst]
        h = jax.ops.segment_sum(norm1[:, None] * jnp.take(xw, src, axis=0),
                                dst, num_segments=_N) + b1
        agg = jax.ops.segment_max(jnp.take(h, src, axis=0), dst, num_segments=_N)
        h = jnp.maximum(h, agg)
        res = h
        hw = h @ W2
        norm2 = d2v[src] * d2v[dst]
        h = jax.ops.segment_sum(norm2[:, None] * jnp.take(hw, src, axis=0),
                                dst, num_segments=_N) + b2
        h = jax.nn.relu(res + h)
        g = jax.ops.segment_max(h, batch, num_segments=_NG)
        return _k3_mlp(jnp.broadcast_to(g, (_NW, _NG, 16)),
                       lin1_W, lin1_b, lin2_W, lin2_b, lin3_W, lin3_b)
    h1 = _p3_conv1(srcP, dstP, wP, meta, xsc, dinv1, b1)
    hmax = _p4_maxpool(srcP, dstP, meta, h1)
    hsc = _k2_scale(hmax, dinv2c, W2)
    batch_pad = jnp.zeros((_NPAD,), jnp.int32).at[:_N].set(batch)
    gpart = _p5_conv2(srcP, dstP, meta, hsc, hmax, dinv2, b2, batch_pad)
    return _k3_mlp(gpart, lin1_W, lin1_b, lin2_W, lin2_b, lin3_W, lin3_b)


# speedup vs baseline: 10.0370x; 1.0078x over previous
import functools

import jax
import jax.numpy as jnp
from jax import lax
from jax.experimental import pallas as pl
from jax.experimental.pallas import tpu as pltpu
from jax.experimental.pallas import tpu_sc as plsc

_N = 100000
_E = 3200000
_NG = 512
_NW = 32
_BW = 3200
_NPAD = _NW * _BW
_NLOC = _BW + 16
_EC = _E // _NW
_EPAD = _E + _NW * 128
_RRELU_SLOPE = (1.0 / 8.0 + 1.0 / 3.0) / 2.0
_CNT_ADJ = 1
_INCREMENT_A = False

_MESH = plsc.VectorSubcoreMesh(core_axis_name="c", subcore_axis_name="s")


def _wid():
    return lax.axis_index("s") * 2 + lax.axis_index("c")


def _bucket_of(dv):
    return ((dv >> 7) * 2622) >> 16


_Z16 = lambda: jnp.zeros((16,), jnp.int32)


def _p1_hist(dst):
    @functools.partial(
        pl.kernel,
        out_type=jax.ShapeDtypeStruct((_NW, 32), jnp.int32),
        mesh=_MESH,
        compiler_params=pltpu.CompilerParams(needs_layout_passes=False, use_tc_tiling_on_sc=False),
        scratch_types=[
            pltpu.VMEM((2000,), jnp.int32),
            pltpu.VMEM((32,), jnp.int32),
        ],
    )
    def k(dst_h, hist_h, dbuf, histv):
        t = _wid()
        histv[pl.ds(0, 16)] = _Z16()
        histv[pl.ds(16, 16)] = _Z16()

        def chunk(c, _):
            pltpu.sync_copy(dst_h.at[pl.ds(t * _EC + c * 2000, 2000)], dbuf)

            def grp(g, _):
                bv = _bucket_of(dbuf[pl.ds(g * 16, 16)])
                cnt, lastm = plsc.scan_count(bv)
                hv = plsc.load_gather(histv, [bv])
                plsc.store_scatter(histv, [bv], hv + cnt + (1 - _CNT_ADJ),
                                   mask=lastm)
                return _
            lax.fori_loop(0, 125, grp, None)
            return _
        lax.fori_loop(0, _EC // 2000, chunk, None)
        pltpu.sync_copy(histv, hist_h.at[t])

    return k(dst)


def _p2_partition(src, dst, w, hist):
    @functools.partial(
        pl.kernel,
        out_type=(
            jax.ShapeDtypeStruct((_EPAD,), jnp.int32),
            jax.ShapeDtypeStruct((_EPAD,), jnp.int32),
            jax.ShapeDtypeStruct((_EPAD,), jnp.float32),
            jax.ShapeDtypeStruct((_NW, 16), jnp.int32),
        ),
        mesh=_MESH,
        compiler_params=pltpu.CompilerParams(needs_layout_passes=False, use_tc_tiling_on_sc=False),
        scratch_types=[
            pltpu.VMEM((_NW, 32), jnp.int32),
            pltpu.VMEM((2000,), jnp.int32),
            pltpu.VMEM((2000,), jnp.int32),
            pltpu.VMEM((2000,), jnp.float32),
            pltpu.VMEM((25, 80), jnp.int32),
            pltpu.VMEM((32,), jnp.int32),
            pltpu.VMEM((16,), jnp.int32),
            pltpu.SemaphoreType.DMA,
        ],
    )
    def k(src_h, dst_h, w_h, hist_h, srcP, dstP, wP, meta_h,
          hbuf, sbuf, dbuf, wbuf, posb, curv_ref, mbuf, sem):
        t = _wid()
        pltpu.sync_copy(hist_h, hbuf)
        z = _Z16()

        def csum(tt, carry):
            lo, hi = carry
            return (lo + hbuf[tt, pl.ds(0, 16)], hi + hbuf[tt, pl.ds(16, 16)])
        cs_lo, cs_hi = lax.fori_loop(0, 32, csum, (z, z))

        pad_lo = ((cs_lo + 127) >> 7) << 7
        pad_hi = ((cs_hi + 127) >> 7) << 7
        inc_lo = plsc.cumsum(pad_lo)
        inc_hi = plsc.cumsum(pad_hi)
        off_lo = inc_lo - pad_lo
        off_hi = inc_hi - pad_hi + inc_lo[15]

        def psum(tt, carry):
            lo, hi = carry
            take = tt < t
            lo = lo + jnp.where(take, hbuf[tt, pl.ds(0, 16)], z)
            hi = hi + jnp.where(take, hbuf[tt, pl.ds(16, 16)], z)
            return (lo, hi)
        p_lo, p_hi = lax.fori_loop(0, 32, psum, (z, z))

        curv_ref[pl.ds(0, 16)] = off_lo + p_lo
        curv_ref[pl.ds(16, 16)] = off_hi + p_hi

        i16 = lax.iota(jnp.int32, 16)
        offt = (jnp.sum(jnp.where(i16 == t, off_lo, z))
                + jnp.sum(jnp.where(i16 == t - 16, off_hi, z)))
        cntt = (jnp.sum(jnp.where(i16 == t, cs_lo, z))
                + jnp.sum(jnp.where(i16 == t - 16, cs_hi, z)))
        mbuf[...] = jnp.where(i16 == 0, offt, z) + jnp.where(i16 == 1, cntt, z)
        pltpu.sync_copy(mbuf, meta_h.at[t])

        def block(c, _):
            s0 = t * _EC + c * 2000
            l1 = pltpu.async_copy(src_h.at[pl.ds(s0, 2000)], sbuf, sem)
            l2 = pltpu.async_copy(dst_h.at[pl.ds(s0, 2000)], dbuf, sem)
            l3 = pltpu.async_copy(w_h.at[pl.ds(s0, 2000)], wbuf, sem)
            l1.wait()
            l2.wait()
            l3.wait()
            scat = []
            for k in range(25):
                for g in range(5):
                    bv = _bucket_of(dbuf[pl.ds(k * 80 + g * 16, 16)])
                    cnt, lastm = plsc.scan_count(bv)
                    curv = plsc.load_gather(curv_ref, [bv])
                    posv = curv + cnt - _CNT_ADJ
                    plsc.store_scatter(curv_ref, [bv], posv + 1, mask=lastm)
                    posb[k, pl.ds(g * 16, 16)] = jnp.clip(posv, 0, _EPAD - 1)
                pr = posb.at[k]
                sl = pl.ds(k * 80, 80)
                scat.append(pltpu.async_copy(sbuf.at[sl], srcP.at[pr], sem))
                scat.append(pltpu.async_copy(dbuf.at[sl], dstP.at[pr], sem))
                scat.append(pltpu.async_copy(wbuf.at[sl], wP.at[pr], sem))
            for d in scat:
                d.wait()
            return _
        lax.fori_loop(0, _EC // 2000, block, None)

    return k(src, dst, w, hist)


def _p25_deg(dstP, wP, meta):
    @functools.partial(
        pl.kernel,
        out_type=(
            jax.ShapeDtypeStruct((_NPAD, 16), jnp.float32),
            jax.ShapeDtypeStruct((_NPAD, 16), jnp.float32),
        ),
        mesh=_MESH,
        compiler_params=pltpu.CompilerParams(needs_layout_passes=False, use_tc_tiling_on_sc=False),
        scratch_types=[
            pltpu.VMEM((16,), jnp.int32),
            pltpu.VMEM((128,), jnp.int32),
            pltpu.VMEM((128,), jnp.float32),
            pltpu.VMEM((_NLOC, 16), jnp.float32),
            pltpu.VMEM((_NLOC, 16), jnp.float32),
        ],
    )
    def k(dstP, wP, meta_h, deg1_h, deg2_h, mbuf, dbuf, wbuf, d1, d2):
        t = _wid()
        base = t * _BW
        pltpu.sync_copy(meta_h.at[t], mbuf)
        mv = mbuf[...]
        off = pl.multiple_of(mv[0], 128)
        cnt = mv[1]

        zf = jnp.zeros((16,), jnp.float32)
        onef = jnp.ones((16,), jnp.float32)

        def zrow(i, _):
            d1[i] = zf
            d2[i] = zf
            return _
        lax.fori_loop(0, _NLOC, zrow, None)

        i16 = lax.iota(jnp.int32, 16)
        nch = (cnt + 127) >> 7

        def chunk(c, _):
            s0 = off + c * 128
            pltpu.sync_copy(dstP.at[pl.ds(s0, 128)], dbuf)
            pltpu.sync_copy(wP.at[pl.ds(s0, 128)], wbuf)

            def grp(g, _):
                m = (c * 128 + g * 16 + i16) < cnt
                dl = jnp.clip(dbuf[pl.ds(g * 16, 16)] - base, 0, _BW)
                dl = jnp.where(m, dl, _BW)
                wv = jnp.where(m, wbuf[pl.ds(g * 16, 16)], 0.0)
                for i in range(16):
                    d = dl[i]
                    plsc.addupdate(d1.at[d], jnp.full((16,), 1.0) * wv[i])
                    plsc.addupdate(d2.at[d], onef)
                return _
            lax.fori_loop(0, 8, grp, None)
            return _
        lax.fori_loop(0, nch, chunk, None)

        pltpu.sync_copy(d1.at[pl.ds(0, _BW)], deg1_h.at[pl.ds(base, _BW)])
        pltpu.sync_copy(d2.at[pl.ds(0, _BW)], deg2_h.at[pl.ds(base, _BW)])

    return k(dstP, wP, meta)


def _p3_conv1(srcP, dstP, wP, meta, xsc, dinv1, b1):
    @functools.partial(
        pl.kernel,
        out_type=jax.ShapeDtypeStruct((_NPAD, 16), jnp.float32),
        mesh=_MESH,
        compiler_params=pltpu.CompilerParams(needs_layout_passes=False, use_tc_tiling_on_sc=False),
        scratch_types=[
            pltpu.VMEM((16,), jnp.int32),
            pltpu.VMEM((128,), jnp.int32),
            pltpu.VMEM((128,), jnp.int32),
            pltpu.VMEM((128,), jnp.float32),
            pltpu.VMEM((128,), jnp.int32),
            pltpu.VMEM((128, 16), jnp.float32),
            pltpu.VMEM((_NLOC, 16), jnp.float32),
            pltpu.VMEM((_BW,), jnp.float32),
            pltpu.VMEM((16,), jnp.float32),
            pltpu.SemaphoreType.DMA,
        ],
    )
    def k(srcP, dstP, wP, meta_h, xsc_h, dinv1_h, b1_h, h1_h,
          mbuf, sbuf, dbuf, wbuf, ibuf, rows, acc, dloc, b1v, sem):
        t = _wid()
        base = t * _BW
        pltpu.sync_copy(meta_h.at[t], mbuf)
        mv = mbuf[...]
        off = pl.multiple_of(mv[0], 128)
        cnt = mv[1]
        pltpu.sync_copy(dinv1_h.at[pl.ds(base, _BW)], dloc)
        pltpu.sync_copy(b1_h, b1v)

        zf = jnp.zeros((16,), jnp.float32)

        def zrow(i, _):
            acc[i] = zf
            return _
        lax.fori_loop(0, _NLOC, zrow, None)

        i16 = lax.iota(jnp.int32, 16)
        nch = (cnt + 127) >> 7

        def chunk(c, _):
            s0 = off + c * 128
            pltpu.sync_copy(srcP.at[pl.ds(s0, 128)], sbuf)
            pltpu.sync_copy(dstP.at[pl.ds(s0, 128)], dbuf)
            pltpu.sync_copy(wP.at[pl.ds(s0, 128)], wbuf)

            def gmask(g, _):
                m = (c * 128 + g * 16 + i16) < cnt
                ibuf[pl.ds(g * 16, 16)] = jnp.clip(
                    jnp.where(m, sbuf[pl.ds(g * 16, 16)], 0), 0, _N - 1)
                return _
            lax.fori_loop(0, 8, gmask, None)
            pltpu.async_copy(xsc_h.at[ibuf], rows, sem).wait()

            def grp(g, _):
                m = (c * 128 + g * 16 + i16) < cnt
                dl = jnp.where(m, jnp.clip(dbuf[pl.ds(g * 16, 16)] - base, 0, _BW), _BW)
                wv = jnp.where(m, wbuf[pl.ds(g * 16, 16)], 0.0)
                for i in range(16):
                    plsc.addupdate(acc.at[dl[i]], rows[g * 16 + i] * wv[i])
                return _
            lax.fori_loop(0, 8, grp, None)
            return _
        lax.fori_loop(0, nch, chunk, None)

        b1vv = b1v[...]

        def nrow(j, _):
            dv = dloc[pl.ds(j * 16, 16)]
            for i in range(16):
                r = j * 16 + i
                acc[r] = acc[r] * dv[i] + b1vv
            return _
        lax.fori_loop(0, _BW // 16, nrow, None)
        pltpu.sync_copy(acc.at[pl.ds(0, _BW)], h1_h.at[pl.ds(base, _BW)])

    return k(srcP, dstP, wP, meta, xsc, dinv1, b1)


def _p4_maxpool(srcP, dstP, meta, h1):
    @functools.partial(
        pl.kernel,
        out_type=jax.ShapeDtypeStruct((_NPAD, 16), jnp.float32),
        mesh=_MESH,
        compiler_params=pltpu.CompilerParams(needs_layout_passes=False, use_tc_tiling_on_sc=False),
        scratch_types=[
            pltpu.VMEM((16,), jnp.int32),
            pltpu.VMEM((128,), jnp.int32),
            pltpu.VMEM((128,), jnp.int32),
            pltpu.VMEM((128,), jnp.int32),
            pltpu.VMEM((128, 16), jnp.float32),
            pltpu.VMEM((_NLOC, 16), jnp.float32),
            pltpu.VMEM((_BW, 16), jnp.float32),
            pltpu.SemaphoreType.DMA,
        ],
    )
    def k(srcP, dstP, meta_h, h1_h, hmax_h,
          mbuf, sbuf, dbuf, ibuf, rows, acc, hown, sem):
        t = _wid()
        base = t * _BW
        pltpu.sync_copy(meta_h.at[t], mbuf)
        mv = mbuf[...]
        off = pl.multiple_of(mv[0], 128)
        cnt = mv[1]
        pltpu.sync_copy(h1_h.at[pl.ds(base, _BW)], hown)

        ninf = jnp.full((16,), -jnp.inf, jnp.float32)

        def irow(i, _):
            acc[i] = ninf
            return _
        lax.fori_loop(0, _NLOC, irow, None)

        i16 = lax.iota(jnp.int32, 16)
        nch = (cnt + 127) >> 7

        def chunk(c, _):
            s0 = off + c * 128
            pltpu.sync_copy(srcP.at[pl.ds(s0, 128)], sbuf)
            pltpu.sync_copy(dstP.at[pl.ds(s0, 128)], dbuf)

            def gmask(g, _):
                m = (c * 128 + g * 16 + i16) < cnt
                ibuf[pl.ds(g * 16, 16)] = jnp.clip(
                    jnp.where(m, sbuf[pl.ds(g * 16, 16)], 0), 0, _N - 1)
                return _
            lax.fori_loop(0, 8, gmask, None)
            pltpu.async_copy(h1_h.at[ibuf], rows, sem).wait()

            def grp(g, _):
                m = (c * 128 + g * 16 + i16) < cnt
                dl = jnp.where(m, jnp.clip(dbuf[pl.ds(g * 16, 16)] - base, 0, _BW), _BW)
                for i in range(16):
                    d = dl[i]
                    acc[d] = jnp.maximum(acc[d], rows[g * 16 + i])
                return _
            lax.fori_loop(0, 8, grp, None)
            return _
        lax.fori_loop(0, nch, chunk, None)

        def nrow(r, _):
            acc[r] = jnp.maximum(acc[r], hown[r])
            return _
        lax.fori_loop(0, _BW, nrow, None)
        pltpu.sync_copy(acc.at[pl.ds(0, _BW)], hmax_h.at[pl.ds(base, _BW)])

    return k(srcP, dstP, meta, h1)


def _p5_conv2(srcP, dstP, meta, hsc, hmax, dinv2, b2, batch_pad):
    @functools.partial(
        pl.kernel,
        out_type=jax.ShapeDtypeStruct((_NW, _NG, 16), jnp.float32),
        mesh=_MESH,
        compiler_params=pltpu.CompilerParams(needs_layout_passes=False, use_tc_tiling_on_sc=False),
        scratch_types=[
            pltpu.VMEM((16,), jnp.int32),
            pltpu.VMEM((128,), jnp.int32),
            pltpu.VMEM((128,), jnp.int32),
            pltpu.VMEM((128,), jnp.int32),
            pltpu.VMEM((128, 16), jnp.float32),
            pltpu.VMEM((_NLOC, 16), jnp.float32),
            pltpu.VMEM((_BW, 16), jnp.float32),
            pltpu.VMEM((_BW,), jnp.float32),
            pltpu.VMEM((_BW,), jnp.int32),
            pltpu.VMEM((_NG + 16, 16), jnp.float32),
            pltpu.VMEM((16,), jnp.float32),
            pltpu.SemaphoreType.DMA,
        ],
    )
    def k(srcP, dstP, meta_h, hsc_h, hmax_h, dinv2_h, b2_h, batch_h, gpart_h,
          mbuf, sbuf, dbuf, ibuf, rows, acc, hown, d2loc, bloc, gacc, b2v, sem):
        t = _wid()
        base = t * _BW
        pltpu.sync_copy(meta_h.at[t], mbuf)
        mv = mbuf[...]
        off = pl.multiple_of(mv[0], 128)
        cnt = mv[1]
        pltpu.sync_copy(hmax_h.at[pl.ds(base, _BW)], hown)
        pltpu.sync_copy(dinv2_h.at[pl.ds(base, _BW)], d2loc)
        pltpu.sync_copy(batch_h.at[pl.ds(base, _BW)], bloc)
        pltpu.sync_copy(b2_h, b2v)

        zf = jnp.zeros((16,), jnp.float32)
        ninf = jnp.full((16,), -jnp.inf, jnp.float32)

        def zrow(i, _):
            acc[i] = zf
            return _
        lax.fori_loop(0, _NLOC, zrow, None)

        def irow(i, _):
            gacc[i] = ninf
            return _
        lax.fori_loop(0, _NG + 16, irow, None)

        i16 = lax.iota(jnp.int32, 16)
        nch = (cnt + 127) >> 7

        def chunk(c, _):
            s0 = off + c * 128
            pltpu.sync_copy(srcP.at[pl.ds(s0, 128)], sbuf)
            pltpu.sync_copy(dstP.at[pl.ds(s0, 128)], dbuf)

            def gmask(g, _):
                m = (c * 128 + g * 16 + i16) < cnt
                ibuf[pl.ds(g * 16, 16)] = jnp.clip(
                    jnp.where(m, sbuf[pl.ds(g * 16, 16)], 0), 0, _N - 1)
                return _
            lax.fori_loop(0, 8, gmask, None)
            pltpu.async_copy(hsc_h.at[ibuf], rows, sem).wait()

            def grp(g, _):
                m = (c * 128 + g * 16 + i16) < cnt
                dl = jnp.where(m, jnp.clip(dbuf[pl.ds(g * 16, 16)] - base, 0, _BW), _BW)
                for i in range(16):
                    plsc.addupdate(acc.at[dl[i]], rows[g * 16 + i])
                return _
            lax.fori_loop(0, 8, grp, None)
            return _
        lax.fori_loop(0, nch, chunk, None)

        b2vv = b2v[...]
        nreal = jnp.minimum(_BW, _N - base)

        def nrow(j, _):
            dv = d2loc[pl.ds(j * 16, 16)]
            bv = bloc[pl.ds(j * 16, 16)]
            for i in range(16):
                r = j * 16 + i
                h2 = jnp.maximum(hown[r] + acc[r] * dv[i] + b2vv, 0.0)
                gi = jnp.where(r < nreal, jnp.clip(bv[i], 0, _NG - 1), _NG)
                gacc[gi] = jnp.maximum(gacc[gi], h2)
            return _
        lax.fori_loop(0, _BW // 16, nrow, None)
        pltpu.sync_copy(gacc.at[pl.ds(0, _NG)], gpart_h.at[t])

    return k(srcP, dstP, meta, hsc, hmax, dinv2, b2, batch_pad)


def _dinv(d):
    return jnp.where(d > 0, lax.rsqrt(jnp.where(d > 0, d, 1.0)), 0.0)


def _k1_body(x_ref, d1_ref, d2_ref, w1_ref, xsc_ref, di1_ref, di2_ref):
    di1 = _dinv(d1_ref[:, 0:1])
    di2 = _dinv(d2_ref[:, 0:1])
    xw = jnp.dot(x_ref[...], w1_ref[...], preferred_element_type=jnp.float32)
    xsc_ref[...] = di1 * xw
    di1_ref[...] = di1
    di2_ref[...] = di2


def _k1_scale(xpad, deg1, deg2, W1p):
    blk = 2048
    grid = _NPAD // blk
    return pl.pallas_call(
        _k1_body,
        grid=(grid,),
        in_specs=[
            pl.BlockSpec((blk, 8), lambda i: (i, 0)),
            pl.BlockSpec((blk, 16), lambda i: (i, 0)),
            pl.BlockSpec((blk, 16), lambda i: (i, 0)),
            pl.BlockSpec((8, 16), lambda i: (0, 0)),
        ],
        out_specs=[
            pl.BlockSpec((blk, 16), lambda i: (i, 0)),
            pl.BlockSpec((blk, 1), lambda i: (i, 0)),
            pl.BlockSpec((blk, 1), lambda i: (i, 0)),
        ],
        out_shape=[
            jax.ShapeDtypeStruct((_NPAD, 16), jnp.float32),
            jax.ShapeDtypeStruct((_NPAD, 1), jnp.float32),
            jax.ShapeDtypeStruct((_NPAD, 1), jnp.float32),
        ],
    )(xpad, deg1, deg2, W1p)


def _k2_body(h_ref, di2_ref, w2_ref, hsc_ref):
    hw = jnp.dot(h_ref[...], w2_ref[...], preferred_element_type=jnp.float32)
    hsc_ref[...] = di2_ref[...] * hw


def _k2_scale(hmax, dinv2c, W2):
    blk = 2048
    grid = _NPAD // blk
    return pl.pallas_call(
        _k2_body,
        grid=(grid,),
        in_specs=[
            pl.BlockSpec((blk, 16), lambda i: (i, 0)),
            pl.BlockSpec((blk, 1), lambda i: (i, 0)),
            pl.BlockSpec((16, 16), lambda i: (0, 0)),
        ],
        out_specs=pl.BlockSpec((blk, 16), lambda i: (i, 0)),
        out_shape=jax.ShapeDtypeStruct((_NPAD, 16), jnp.float32),
    )(hmax, dinv2c, W2)


def _rrelu(x):
    return jnp.where(x >= 0, x, x * _RRELU_SLOPE)


def _k3_body(gp_ref, w1_ref, b1_ref, w2_ref, b2_ref, w3_ref, b3_ref, out_ref):
    g = jnp.max(gp_ref[...], axis=0)
    g = _rrelu(g + jnp.dot(g, w1_ref[...], preferred_element_type=jnp.float32) + b1_ref[...])
    g = _rrelu(g + jnp.dot(g, w2_ref[...], preferred_element_type=jnp.float32) + b2_ref[...])
    g = _rrelu(jnp.dot(g, w3_ref[...], preferred_element_type=jnp.float32) + b3_ref[...])
    out_ref[...] = g


def _k3_mlp(gpart, lin1_W, lin1_b, lin2_W, lin2_b, lin3_W, lin3_b):
    return pl.pallas_call(
        _k3_body,
        out_shape=jax.ShapeDtypeStruct((_NG, 1), jnp.float32),
    )(gpart, lin1_W, lin1_b.reshape(1, -1), lin2_W, lin2_b.reshape(1, -1),
      lin3_W, lin3_b.reshape(1, -1))


def kernel(x, edge_index, batch, edge_weights, W1, b1, W2, b2,
           lin1_W, lin1_b, lin2_W, lin2_b, lin3_W, lin3_b):
    src = edge_index[0]
    dst = edge_index[1]

    hist = _p1_hist(dst)
    srcP, dstP, wP, meta = _p2_partition(src, dst, edge_weights, hist)
    deg1p, deg2p = _p25_deg(dstP, wP, meta)

    xpad = jnp.zeros((_NPAD, 8), jnp.float32).at[:_N, :7].set(x)
    W1p = jnp.zeros((8, 16), jnp.float32).at[:7].set(W1)
    xsc, dinv1c, dinv2c = _k1_scale(xpad, deg1p, deg2p, W1p)
    dinv1 = dinv1c.reshape(_NPAD)
    dinv2 = dinv2c.reshape(_NPAD)

    if _INCREMENT_A:
        d1v = dinv1[:_N]
        d2v = dinv2[:_N]
        xw = x @ W1
        norm1 = d1v[src] * edge_weights * d1v[dst]
        h = jax.ops.segment_sum(norm1[:, None] * jnp.take(xw, src, axis=0),
                                dst, num_segments=_N) + b1
        agg = jax.ops.segment_max(jnp.take(h, src, axis=0), dst, num_segments=_N)
        h = jnp.maximum(h, agg)
        res = h
        hw = h @ W2
        norm2 = d2v[src] * d2v[dst]
        h = jax.ops.segment_sum(norm2[:, None] * jnp.take(hw, src, axis=0),
                                dst, num_segments=_N) + b2
        h = jax.nn.relu(res + h)
        g = jax.ops.segment_max(h, batch, num_segments=_NG)
        return _k3_mlp(jnp.broadcast_to(g, (_NW, _NG, 16)),
                       lin1_W, lin1_b, lin2_W, lin2_b, lin3_W, lin3_b)
    h1 = _p3_conv1(srcP, dstP, wP, meta, xsc, dinv1, b1)
    hmax = _p4_maxpool(srcP, dstP, meta, h1)
    hsc = _k2_scale(hmax, dinv2c, W2)
    batch_pad = jnp.zeros((_NPAD,), jnp.int32).at[:_N].set(batch)
    gpart = _p5_conv2(srcP, dstP, meta, hsc, hmax, dinv2, b2, batch_pad)
    return _k3_mlp(gpart, lin1_W, lin1_b, lin2_W, lin2_b, lin3_W, lin3_b)
